# Initial kernel scaffold; baseline (speedup 1.0000x reference)
import jax
import jax.numpy as jnp
from jax.experimental import pallas as pl


def kernel(x, t, table, W1, b1, W2, b2):
    raise NotImplementedError("write your pallas kernel here")


# speedup vs baseline: 1.4827x; 1.4827x over previous
import functools
import math

import jax
import jax.numpy as jnp
from jax import lax
from jax.experimental import pallas as pl
from jax.experimental.pallas import tpu as pltpu
from jax.experimental.pallas import tpu_sc as plsc

_DIM = 2048
_NUM_HEADS = 16
_NC = 2
_NS = 16
_NW = _NC * _NS
_CH = 16


def _sc_gather(idx, table):
    nw, n_chunks, ch = idx.shape
    _, d = table.shape
    b = nw * n_chunks * ch
    rows_per_w = n_chunks * ch
    mesh = plsc.VectorSubcoreMesh(core_axis_name="c", subcore_axis_name="s")

    @functools.partial(
        pl.kernel,
        out_type=jax.ShapeDtypeStruct((b, d), jnp.float32),
        mesh=mesh,
        scratch_types=[
            pltpu.VMEM((n_chunks, ch), jnp.int32),
            pltpu.VMEM((ch, d), jnp.float32),
            pltpu.SemaphoreType.DMA,
        ],
    )
    def gather_kernel(idx_hbm, table_hbm, out_hbm, idx_v, buf, sem):
        wid = lax.axis_index("s") * _NC + lax.axis_index("c")
        base = wid * rows_per_w
        pltpu.sync_copy(idx_hbm.at[wid], idx_v)

        @pl.loop(0, n_chunks)
        def _(j):
            pltpu.async_copy(table_hbm.at[idx_v.at[j]], buf, sem).wait()
            pltpu.sync_copy(buf, out_hbm.at[pl.ds(base + j * ch, ch)])

    return gather_kernel(idx, table)


def _tc_body(t_ref, w1_ref, b1_ref, w2_ref, b2_ref, c_ref, pos_ref):
    half = _DIM // 2
    i = lax.broadcasted_iota(jnp.int32, (4, half), 1).astype(jnp.float32)
    freqs = jnp.exp(i * (-math.log(10000.0) / half))
    args = t_ref[...] * freqs
    emb = jnp.concatenate([jnp.sin(args), jnp.cos(args)], axis=-1)
    h = jnp.dot(emb, w1_ref[...], preferred_element_type=jnp.float32)
    h = h + b1_ref[...]
    h = h * (1.0 / (1.0 + jnp.exp(-h)))
    c = jnp.dot(h, w2_ref[...], preferred_element_type=jnp.float32)
    c_ref[...] = c + b2_ref[...]

    head_dim = _DIM // _NUM_HEADS
    hh = head_dim // 2
    ln = pos_ref.shape[1]
    p = lax.broadcasted_iota(jnp.int32, (ln, hh), 0).astype(jnp.float32)
    fi = lax.broadcasted_iota(jnp.int32, (ln, hh), 1).astype(jnp.float32)
    inv_freq = jnp.exp(fi * (-2.0 * math.log(10000.0) / head_dim))
    fr = p * inv_freq
    emb2 = jnp.concatenate([fr, fr], axis=-1)
    pos_ref[0] = jnp.cos(emb2)
    pos_ref[1] = jnp.sin(emb2)


def _tc_mlp_rotary(t, w1, b1, w2, b2, seq_len):
    head_dim = _DIM // _NUM_HEADS
    return pl.pallas_call(
        _tc_body,
        out_shape=(
            jax.ShapeDtypeStruct((4, _DIM), jnp.float32),
            jax.ShapeDtypeStruct((2, seq_len, head_dim), jnp.float32),
        ),
    )(t, w1, b1, w2, b2)


def kernel(x, t, table, W1, b1, W2, b2):
    batch, seq_len = x.shape
    n_rows = batch * seq_len
    n_chunks = n_rows // (_NW * _CH)
    idx = x.astype(jnp.int32).reshape(_NW, n_chunks, _CH)
    h = _sc_gather(idx, table).reshape(batch, seq_len, _DIM)
    c, pos = _tc_mlp_rotary(
        t.reshape(4, 1), W1, b1.reshape(1, _DIM), W2, b2.reshape(1, _DIM), seq_len
    )
    return (h, c, pos)


# speedup vs baseline: 1.6752x; 1.1299x over previous
import functools
import math

import jax
import jax.numpy as jnp
from jax import lax
from jax.experimental import pallas as pl
from jax.experimental.pallas import tpu as pltpu
from jax.experimental.pallas import tpu_sc as plsc

_DIM = 2048
_NUM_HEADS = 16
_NC = 2
_NS = 16
_NW = _NC * _NS
_CH = 8
_NB = 4


def _sc_gather(idx, table):
    nw, n_chunks, ch = idx.shape
    _, d = table.shape
    b = nw * n_chunks * ch
    rows_per_w = n_chunks * ch
    assert n_chunks % _NB == 0
    mesh = plsc.VectorSubcoreMesh(core_axis_name="c", subcore_axis_name="s")

    @functools.partial(
        pl.kernel,
        out_type=jax.ShapeDtypeStruct((b, d), jnp.float32),
        mesh=mesh,
        scratch_types=[
            pltpu.VMEM((n_chunks, ch), jnp.int32),
            [pltpu.VMEM((ch, d), jnp.float32) for _ in range(_NB)],
            [pltpu.SemaphoreType.DMA for _ in range(_NB)],
            [pltpu.SemaphoreType.DMA for _ in range(_NB)],
        ],
    )
    def gather_kernel(idx_hbm, table_hbm, out_hbm, idx_v, bufs, gsems, wsems):
        wid = lax.axis_index("s") * _NC + lax.axis_index("c")
        base = wid * rows_per_w
        pltpu.sync_copy(idx_hbm.at[wid], idx_v)

        for bi in range(_NB):
            pltpu.async_copy(table_hbm.at[idx_v.at[bi]], bufs[bi], gsems[bi])

        @pl.loop(0, n_chunks, step=_NB)
        def _(j0):
            for bi in range(_NB):
                pltpu.make_async_copy(
                    table_hbm.at[idx_v.at[0]], bufs[bi], gsems[bi]
                ).wait()
                pltpu.async_copy(
                    bufs[bi], out_hbm.at[pl.ds(base + (j0 + bi) * ch, ch)],
                    wsems[bi],
                )

            @pl.when(j0 + _NB < n_chunks)
            def _():
                for bi in range(_NB):
                    pltpu.make_async_copy(
                        bufs[bi], out_hbm.at[pl.ds(0, ch)], wsems[bi]
                    ).wait()
                    pltpu.async_copy(
                        table_hbm.at[idx_v.at[j0 + _NB + bi]], bufs[bi],
                        gsems[bi],
                    )

        for bi in range(_NB):
            pltpu.make_async_copy(
                bufs[bi], out_hbm.at[pl.ds(0, ch)], wsems[bi]
            ).wait()

    return gather_kernel(idx, table)


def _tc_body(t_ref, w1_ref, b1_ref, w2_ref, b2_ref, c_ref, pos_ref):
    half = _DIM // 2
    i = lax.broadcasted_iota(jnp.int32, (4, half), 1).astype(jnp.float32)
    freqs = jnp.exp(i * (-math.log(10000.0) / half))
    args = t_ref[...] * freqs
    emb = jnp.concatenate([jnp.sin(args), jnp.cos(args)], axis=-1)
    h = jnp.dot(emb, w1_ref[...], preferred_element_type=jnp.float32)
    h = h + b1_ref[...]
    h = h * (1.0 / (1.0 + jnp.exp(-h)))
    c = jnp.dot(h, w2_ref[...], preferred_element_type=jnp.float32)
    c_ref[...] = c + b2_ref[...]

    head_dim = _DIM // _NUM_HEADS
    hh = head_dim // 2
    ln = pos_ref.shape[1]
    p = lax.broadcasted_iota(jnp.int32, (ln, hh), 0).astype(jnp.float32)
    fi = lax.broadcasted_iota(jnp.int32, (ln, hh), 1).astype(jnp.float32)
    inv_freq = jnp.exp(fi * (-2.0 * math.log(10000.0) / head_dim))
    fr = p * inv_freq
    emb2 = jnp.concatenate([fr, fr], axis=-1)
    pos_ref[0] = jnp.cos(emb2)
    pos_ref[1] = jnp.sin(emb2)


def _tc_mlp_rotary(t, w1, b1, w2, b2, seq_len):
    head_dim = _DIM // _NUM_HEADS
    return pl.pallas_call(
        _tc_body,
        out_shape=(
            jax.ShapeDtypeStruct((4, _DIM), jnp.float32),
            jax.ShapeDtypeStruct((2, seq_len, head_dim), jnp.float32),
        ),
    )(t, w1, b1, w2, b2)


def kernel(x, t, table, W1, b1, W2, b2):
    batch, seq_len = x.shape
    n_rows = batch * seq_len
    n_chunks = n_rows // (_NW * _CH)
    idx = x.astype(jnp.int32).reshape(_NW, n_chunks, _CH)
    h = _sc_gather(idx, table).reshape(batch, seq_len, _DIM)
    c, pos = _tc_mlp_rotary(
        t.reshape(4, 1), W1, b1.reshape(1, _DIM), W2, b2.reshape(1, _DIM), seq_len
    )
    return (h, c, pos)
